# baseline (speedup 1.0000x reference)
import functools

import jax
import jax.numpy as jnp
from jax import lax
from jax.experimental import pallas as pl
from jax.experimental.pallas import tpu as pltpu
from jax.experimental.pallas import tpu_sc as plsc

V0 = 1000000
V1 = 100000
D = 64
NC = 2
NS = 16
LANES = 16
NW = NC * NS

K = 128
NB = 4
P = 128


@functools.lru_cache(maxsize=None)
def _build(N):
    assert N % NW == 0
    chunk = N // NW
    assert chunk % (K * NB) == 0
    nj = chunk // K
    njo = nj // NB
    trash = chunk + P
    m1cap = chunk + P + 16

    mesh = plsc.VectorSubcoreMesh(core_axis_name="c", subcore_axis_name="s")

    @functools.partial(
        pl.kernel,
        out_type=jax.ShapeDtypeStruct((N, D), jnp.float32),
        mesh=mesh,
        compiler_params=pltpu.CompilerParams(
            use_tc_tiling_on_sc=False, needs_layout_passes=False),
        scratch_types=[
            [pltpu.VMEM((K,), jnp.int32)] * NB,
            [pltpu.VMEM((K,), jnp.int32)] * NB,
            [pltpu.VMEM((K, D), jnp.float32)] * NB,
            pltpu.VMEM((m1cap,), jnp.int32),
            pltpu.VMEM((m1cap,), jnp.int32),
            pltpu.VMEM((P,), jnp.int32),
            pltpu.VMEM((P,), jnp.int32),
            pltpu.VMEM((P, D), jnp.float32),
            [pltpu.SemaphoreType.DMA] * NB,
            [pltpu.SemaphoreType.DMA] * NB,
            [pltpu.SemaphoreType.DMA] * NB,
            pltpu.SemaphoreType.DMA,
        ],
    )
    def emb(t0, t1, xf, out, idxv, l0, r0, list1, pos1, plist, ppos, prows,
            semi, semg, semw, semp):
        wid = lax.axis_index("s") * NC + lax.axis_index("c")
        base = wid * chunk

        for b in range(NB):
            pltpu.async_copy(xf.at[pl.ds(base + b * K, K)], idxv[b], semi[b])

        def outer(jo, n1):
            for b in range(NB):
                j = jo * NB + b
                off = base + j * K
                pltpu.make_async_copy(
                    xf.at[pl.ds(off, K)], idxv[b], semi[b]).wait()

                def vec(v, n1_, _b=b, _off=off):
                    sl = pl.ds(v * LANES, LANES)
                    vi = idxv[_b][sl]
                    m1 = vi >= V0
                    l0[_b][sl] = jnp.minimum(vi, V0 - 1)
                    gpos = (_off + v * LANES
                            + lax.broadcasted_iota(jnp.int32, (LANES,), 0))
                    i1 = jnp.clip(vi - V0, 0, V1 - 1)
                    m1i = m1.astype(jnp.int32)
                    excl = plsc.cumsum(m1i) - m1i
                    dst = jnp.where(m1, n1_ + excl, jnp.int32(trash))
                    plsc.store_scatter(list1, [dst], i1)
                    plsc.store_scatter(pos1, [dst], gpos)
                    return n1_ + jnp.sum(m1i)

                n1 = lax.fori_loop(0, K // LANES, vec, n1)

                @pl.when(jo < njo - 1)
                def _():
                    pltpu.async_copy(
                        xf.at[pl.ds(off + NB * K, K)], idxv[b], semi[b])

                @pl.when(jo > 0)
                def _():
                    pltpu.make_async_copy(
                        r0[b], out.at[pl.ds(off - NB * K, K)], semw[b]).wait()

                pltpu.async_copy(t0.at[l0[b]], r0[b], semg[b])

                pb = (b - 1) % NB
                poff = off - K
                @pl.when(j > 0)
                def _():
                    pltpu.make_async_copy(
                        t0.at[l0[pb]], r0[pb], semg[pb]).wait()
                    pltpu.async_copy(
                        r0[pb], out.at[pl.ds(poff, K)], semw[pb])
            return n1

        n1 = lax.fori_loop(0, njo, outer, jnp.int32(0))

        lb = NB - 1
        loff = base + (nj - 1) * K
        pltpu.make_async_copy(t0.at[l0[lb]], r0[lb], semg[lb]).wait()
        pltpu.async_copy(r0[lb], out.at[pl.ds(loff, K)], semw[lb])
        for b in range(NB):
            j = nj - NB + b
            pltpu.make_async_copy(
                r0[b], out.at[pl.ds(base + j * K, K)], semw[b]).wait()

        zero16 = jnp.zeros((LANES,), jnp.int32)
        dup_l = plsc.load_gather(list1, [zero16])
        dup_p = plsc.load_gather(pos1, [zero16])
        for q in range(P // LANES):
            list1[pl.ds(n1 + q * LANES, LANES)] = dup_l
            pos1[pl.ds(n1 + q * LANES, LANES)] = dup_p

        def consume_block(o):
            for q in range(P // LANES):
                sl = pl.ds(q * LANES, LANES)
                plist[sl] = list1[pl.ds(o + q * LANES, LANES)]
                ppos[sl] = pos1[pl.ds(o + q * LANES, LANES)]
            pltpu.async_copy(t1.at[plist], prows, semp).wait()
            pltpu.async_copy(prows, out.at[ppos], semp).wait()

        def tail_left(nd):
            return nd < n1

        def tail_block(nd):
            consume_block(nd)
            return nd + P

        lax.while_loop(tail_left, tail_block, jnp.int32(0))

    return emb


@jax.jit
def kernel(table0, table1, x):
    B, L = x.shape
    n = B * L
    xf = x.reshape(n)
    out = _build(n)(table0, table1, xf)
    return out.reshape(B, L, D)


# speedup vs baseline: 2.2613x; 2.2613x over previous
import functools

import jax
import jax.numpy as jnp
from jax import lax
from jax.experimental import pallas as pl
from jax.experimental.pallas import tpu as pltpu
from jax.experimental.pallas import tpu_sc as plsc

V0 = 1000000
V1 = 100000
D = 64
NC = 2
NS = 16
LANES = 16
NW = NC * NS

K = 128
NB = 4
P = 128


@functools.lru_cache(maxsize=None)
def _build(N):
    assert N % NW == 0
    chunk = N // NW
    assert chunk % (K * NB) == 0
    nj = chunk // K
    njo = nj // NB
    trash = chunk + P
    m1cap = chunk + P + 16

    mesh = plsc.VectorSubcoreMesh(core_axis_name="c", subcore_axis_name="s")

    @functools.partial(
        pl.kernel,
        out_type=jax.ShapeDtypeStruct((N, D), jnp.float32),
        mesh=mesh,
        compiler_params=pltpu.CompilerParams(
            use_tc_tiling_on_sc=False, needs_layout_passes=False),
        scratch_types=[
            [pltpu.VMEM((K,), jnp.int32)] * NB,
            [pltpu.VMEM((K,), jnp.int32)] * NB,
            [pltpu.VMEM((K, D), jnp.float32)] * NB,
            pltpu.VMEM((m1cap,), jnp.int32),
            pltpu.VMEM((m1cap,), jnp.int32),
            pltpu.VMEM((P,), jnp.int32),
            pltpu.VMEM((P,), jnp.int32),
            pltpu.VMEM((P, D), jnp.float32),
            [pltpu.SemaphoreType.DMA] * NB,
            [pltpu.SemaphoreType.DMA] * NB,
            [pltpu.SemaphoreType.DMA] * NB,
            pltpu.SemaphoreType.DMA,
        ],
    )
    def emb(t0, t1, xf, out, idxv, l0, r0, list1, pos1, plist, ppos, prows,
            semi, semg, semw, semp):
        wid = lax.axis_index("s") * NC + lax.axis_index("c")
        base = wid * chunk

        for b in range(NB):
            pltpu.async_copy(xf.at[pl.ds(base + b * K, K)], idxv[b], semi[b])

        def outer(jo, n1):
            for b in range(NB):
                j = jo * NB + b
                off = base + j * K
                pltpu.make_async_copy(
                    xf.at[pl.ds(off, K)], idxv[b], semi[b]).wait()

                def vec(v, n1_, _b=b, _off=off):
                    sl = pl.ds(v * LANES, LANES)
                    vi = idxv[_b][sl]
                    m1 = vi >= V0
                    l0[_b][sl] = jnp.minimum(
                        jnp.where(m1, vi - V0, vi), V0 - 1)
                    gpos = (_off + v * LANES
                            + lax.broadcasted_iota(jnp.int32, (LANES,), 0))
                    i1 = jnp.clip(vi - V0, 0, V1 - 1)
                    m1i = m1.astype(jnp.int32)
                    excl = plsc.cumsum(m1i) - m1i
                    dst = jnp.where(m1, n1_ + excl, jnp.int32(trash))
                    plsc.store_scatter(list1, [dst], i1)
                    plsc.store_scatter(pos1, [dst], gpos)
                    return n1_ + jnp.sum(m1i)

                n1 = lax.fori_loop(0, K // LANES, vec, n1)

                @pl.when(jo < njo - 1)
                def _():
                    pltpu.async_copy(
                        xf.at[pl.ds(off + NB * K, K)], idxv[b], semi[b])

                @pl.when(jo > 0)
                def _():
                    pltpu.make_async_copy(
                        r0[b], out.at[pl.ds(off - NB * K, K)], semw[b]).wait()

                pltpu.async_copy(t0.at[l0[b]], r0[b], semg[b])

                pb = (b - 1) % NB
                poff = off - K
                @pl.when(j > 0)
                def _():
                    pltpu.make_async_copy(
                        t0.at[l0[pb]], r0[pb], semg[pb]).wait()
                    pltpu.async_copy(
                        r0[pb], out.at[pl.ds(poff, K)], semw[pb])
            return n1

        n1 = lax.fori_loop(0, njo, outer, jnp.int32(0))

        lb = NB - 1
        loff = base + (nj - 1) * K
        pltpu.make_async_copy(t0.at[l0[lb]], r0[lb], semg[lb]).wait()
        pltpu.async_copy(r0[lb], out.at[pl.ds(loff, K)], semw[lb])
        for b in range(NB):
            j = nj - NB + b
            pltpu.make_async_copy(
                r0[b], out.at[pl.ds(base + j * K, K)], semw[b]).wait()

        zero16 = jnp.zeros((LANES,), jnp.int32)
        dup_l = plsc.load_gather(list1, [zero16])
        dup_p = plsc.load_gather(pos1, [zero16])
        for q in range(P // LANES):
            list1[pl.ds(n1 + q * LANES, LANES)] = dup_l
            pos1[pl.ds(n1 + q * LANES, LANES)] = dup_p

        def consume_block(o):
            for q in range(P // LANES):
                sl = pl.ds(q * LANES, LANES)
                plist[sl] = list1[pl.ds(o + q * LANES, LANES)]
                ppos[sl] = pos1[pl.ds(o + q * LANES, LANES)]
            pltpu.async_copy(t1.at[plist], prows, semp).wait()
            pltpu.async_copy(prows, out.at[ppos], semp).wait()

        def tail_left(nd):
            return nd < n1

        def tail_block(nd):
            consume_block(nd)
            return nd + P

        lax.while_loop(tail_left, tail_block, jnp.int32(0))

    return emb


@jax.jit
def kernel(table0, table1, x):
    B, L = x.shape
    n = B * L
    xf = x.reshape(n)
    out = _build(n)(table0, table1, xf)
    return out.reshape(B, L, D)
